# baseline (speedup 1.0000x reference)
import functools

import jax
import jax.numpy as jnp
from jax import lax
from jax.experimental import pallas as pl
from jax.experimental.pallas import tpu as pltpu
from jax.experimental.pallas import tpu_sc as plsc

B = 8
N = 2048
D = 128
NUM_QUERIES = 128
NUM_SUPPORTS = N - NUM_QUERIES
K_KEEP = NUM_SUPPORTS // 2
N_OUT = K_KEEP + NUM_QUERIES
N_PAD = 1152

_SUP_BLOCKS = NUM_SUPPORTS // 128

_NW = 32
_WPB = _NW // B
_RPW = N_OUT // _WPB
_CH = 16
_NCHUNK = _RPW // _CH


_JW = 384
_IW = 384


def _score_topk_body(x_ref, w_ref, b_ref, idx_ref, val_ref,
                     sj_ref, si_ref, rank_ref):
    x = x_ref[0]
    w = w_ref[...]
    logits = jnp.dot(x, w, preferred_element_type=jnp.float32)
    scores_col = jax.nn.sigmoid((logits + b_ref[0, 0]) / 100.0)
    scores_row = jnp.reshape(scores_col, (N // 128, 128))

    s_i = scores_col[:NUM_SUPPORTS]
    si_ref[...] = s_i
    sj_ref[...] = jnp.reshape(s_i, (NUM_SUPPORTS // _JW, _JW))
    i_iota = lax.broadcasted_iota(jnp.int32, (NUM_SUPPORTS, 1), 0)

    def cnt_body(t, cnt):
        s_j = sj_ref[pl.ds(t, 1), :]
        j_iota = t * _JW + lax.broadcasted_iota(jnp.int32, (1, _JW), 1)
        gt = s_j > s_i
        eq_lt = (s_j == s_i) & (j_iota < i_iota)
        return cnt + jnp.sum(jnp.where(gt | eq_lt, 1.0, 0.0),
                             axis=1, keepdims=True)

    cnt = lax.fori_loop(0, NUM_SUPPORTS // _JW, cnt_body,
                        jnp.zeros((NUM_SUPPORTS, 1), jnp.float32))
    rank_ref[...] = jnp.where(cnt < K_KEEP, cnt, -1.0)

    r_row = lax.broadcasted_iota(jnp.int32, (1, N_PAD), 1).astype(jnp.float32)

    def scat_body(t, accs):
        idx_acc, val_acc = accs
        rk = rank_ref[pl.ds(t * _IW, _IW), :]
        sv = si_ref[pl.ds(t * _IW, _IW), :]
        iv = (t * _IW
              + lax.broadcasted_iota(jnp.int32, (_IW, 1), 0)).astype(jnp.float32)
        onehot = jnp.where(rk == r_row, 1.0, 0.0)
        idx_acc = idx_acc + jnp.sum(onehot * iv, axis=0, keepdims=True)
        val_acc = val_acc + jnp.sum(onehot * sv, axis=0, keepdims=True)
        return (idx_acc, val_acc)

    idx_acc, val_acc = lax.fori_loop(
        0, NUM_SUPPORTS // _IW, scat_body,
        (jnp.zeros((1, N_PAD), jnp.float32), jnp.zeros((1, N_PAD), jnp.float32)))

    q_scores = scores_row[N // 128 - 1:N // 128, :]
    q_mask = (r_row >= float(K_KEEP)) & (r_row < float(N_OUT))
    q_idx = jnp.where(q_mask, r_row + float(K_KEEP), 0.0)
    q_val_pad = jnp.concatenate(
        [jnp.zeros((1, K_KEEP), jnp.float32), q_scores,
         jnp.zeros((1, N_PAD - N_OUT), jnp.float32)], axis=1)

    idx_ref[...] = jnp.reshape((idx_acc + q_idx).astype(jnp.int32), (1, 1, N_PAD))
    val_ref[...] = jnp.reshape(val_acc + q_val_pad, (1, 1, N_PAD))


def _score_topk(X, W, b):
    w_col = jnp.reshape(W, (D, 1))
    b_mat = jnp.reshape(b, (1, 1))
    idxp, valp = pl.pallas_call(
        _score_topk_body,
        grid=(B,),
        in_specs=[
            pl.BlockSpec((1, N, D), lambda i: (i, 0, 0)),
            pl.BlockSpec((D, 1), lambda i: (0, 0)),
            pl.BlockSpec((1, 1), lambda i: (0, 0)),
        ],
        out_specs=[
            pl.BlockSpec((1, 1, N_PAD), lambda i: (i, 0, 0)),
            pl.BlockSpec((1, 1, N_PAD), lambda i: (i, 0, 0)),
        ],
        out_shape=[
            jax.ShapeDtypeStruct((B, 1, N_PAD), jnp.int32),
            jax.ShapeDtypeStruct((B, 1, N_PAD), jnp.float32),
        ],
        scratch_shapes=[
            pltpu.VMEM((NUM_SUPPORTS // _JW, _JW), jnp.float32),
            pltpu.VMEM((NUM_SUPPORTS, 1), jnp.float32),
            pltpu.VMEM((NUM_SUPPORTS, 1), jnp.float32),
        ],
    )(X, w_col, b_mat)
    return idxp, valp


def _sc_gather_body(a_hbm, x_hbm, idx_hbm, val_hbm, outa_hbm, outx_hbm,
                    colidx_v, rowidx_v, vals_v,
                    abuf0, abuf1, obuf0, obuf1, xbuf0, xbuf1, xobuf0, xobuf1,
                    sem_a0, sem_a1, sem_x0, sem_x1,
                    sem_oa0, sem_oa1, sem_ox0, sem_ox1):
    wid = lax.axis_index("s") * 2 + lax.axis_index("c")
    bi = wid // _WPB
    rb = (wid % _WPB) * _RPW

    base = pl.multiple_of(bi * N_PAD, N_PAD)
    pltpu.sync_copy(idx_hbm.at[pl.ds(base, K_KEEP)], colidx_v)
    pltpu.sync_copy(idx_hbm.at[pl.ds(base + rb, _RPW)], rowidx_v)
    pltpu.sync_copy(val_hbm.at[pl.ds(base + rb, _RPW)], vals_v)

    def rows_of(c):
        return rowidx_v.at[pl.ds(pl.multiple_of(c * _CH, _CH), _CH)]

    def start_in(c, ab, xb, sa, sx):
        pltpu.async_copy(a_hbm.at[bi].at[rows_of(c)], ab, sa)
        pltpu.async_copy(x_hbm.at[bi].at[rows_of(c)], xb, sx)

    def wait_in(c, ab, xb, sa, sx):
        pltpu.make_async_copy(a_hbm.at[bi].at[rows_of(c)], ab, sa).wait()
        pltpu.make_async_copy(x_hbm.at[bi].at[rows_of(c)], xb, sx).wait()

    def out_slices(c):
        c8 = pl.multiple_of(c * _CH, _CH)
        return (outa_hbm.at[bi, pl.ds(rb + c8, _CH), :],
                outx_hbm.at[bi, pl.ds(rb + c8, _CH), :])

    def start_out(c, ob, xob, soa, sox):
        oa, ox = out_slices(c)
        pltpu.async_copy(ob, oa, soa)
        pltpu.async_copy(xob, ox, sox)

    def wait_out(c, ob, xob, soa, sox):
        oa, ox = out_slices(c)
        pltpu.make_async_copy(ob, oa, soa).wait()
        pltpu.make_async_copy(xob, ox, sox).wait()

    def compute(c, ab, xb, ob, xob):
        c8 = pl.multiple_of(c * _CH, _CH)
        for g in range(K_KEEP // 16):
            ci = colidx_v[pl.ds(g * 16, 16)]
            for j in range(_CH):
                ob[j, pl.ds(g * 16, 16)] = plsc.load_gather(ab.at[j], [ci])
        for j in range(_CH):
            for g in range(NUM_QUERIES // 16):
                ob[j, pl.ds(K_KEEP + g * 16, 16)] = \
                    ab[j, pl.ds(NUM_SUPPORTS + g * 16, 16)]
            vsp = plsc.load_gather(vals_v, [jnp.full((16,), 1, jnp.int32)
                                            * (c8 + j)])
            for g in range(D // 16):
                xob[j, pl.ds(g * 16, 16)] = xb[j, pl.ds(g * 16, 16)] * vsp

    bufs = ((abuf0, xbuf0, obuf0, xobuf0, sem_a0, sem_x0, sem_oa0, sem_ox0),
            (abuf1, xbuf1, obuf1, xobuf1, sem_a1, sem_x1, sem_oa1, sem_ox1))

    start_in(0, abuf0, xbuf0, sem_a0, sem_x0)

    def pair(i, carry):
        c0 = i * 2
        for par in range(2):
            ab, xb, ob, xob, sa, sx, soa, sox = bufs[par]
            nab, nxb, _, _, nsa, nsx, _, _ = bufs[1 - par]
            c = c0 + par

            @pl.when(c + 1 < _NCHUNK)
            def _():
                start_in(c + 1, nab, nxb, nsa, nsx)

            wait_in(c, ab, xb, sa, sx)

            @pl.when(c >= 2)
            def _():
                wait_out(c - 2, ob, xob, soa, sox)

            compute(c, ab, xb, ob, xob)
            start_out(c, ob, xob, soa, sox)
        return carry

    lax.fori_loop(0, _NCHUNK // 2, pair, 0)
    if _NCHUNK % 2:
        c = _NCHUNK - 1
        wait_in(c, abuf0, xbuf0, sem_a0, sem_x0)
        wait_out(c - 2, obuf0, xobuf0, sem_oa0, sem_ox0)
        compute(c, abuf0, xbuf0, obuf0, xobuf0)
        start_out(c, obuf0, xobuf0, sem_oa0, sem_ox0)
    wait_out(_NCHUNK - 2, obuf1 if _NCHUNK % 2 else obuf0,
             xobuf1 if _NCHUNK % 2 else xobuf0,
             sem_oa1 if _NCHUNK % 2 else sem_oa0,
             sem_ox1 if _NCHUNK % 2 else sem_ox0)
    wait_out(_NCHUNK - 1, obuf0 if _NCHUNK % 2 else obuf1,
             xobuf0 if _NCHUNK % 2 else xobuf1,
             sem_oa0 if _NCHUNK % 2 else sem_oa1,
             sem_ox0 if _NCHUNK % 2 else sem_ox1)


def _sc_gather(A, X, idxp, valp):
    mesh = plsc.VectorSubcoreMesh(core_axis_name="c", subcore_axis_name="s")
    kern = pl.kernel(
        _sc_gather_body,
        out_type=[
            jax.ShapeDtypeStruct((B, N_OUT, N_OUT), jnp.float32),
            jax.ShapeDtypeStruct((B, N_OUT, D), jnp.float32),
        ],
        mesh=mesh,
        scratch_types=(
            [pltpu.VMEM((K_KEEP,), jnp.int32),
             pltpu.VMEM((_RPW,), jnp.int32),
             pltpu.VMEM((_RPW,), jnp.float32)]
            + [pltpu.VMEM((_CH, N), jnp.float32)] * 2
            + [pltpu.VMEM((_CH, N_OUT), jnp.float32)] * 2
            + [pltpu.VMEM((_CH, D), jnp.float32)] * 4
            + [pltpu.SemaphoreType.DMA] * 8
        ),
        compiler_params=pltpu.CompilerParams(use_tc_tiling_on_sc=False,
                                             needs_layout_passes=False),
    )
    return kern(A, X, jnp.reshape(idxp, (B * N_PAD,)),
                jnp.reshape(valp, (B * N_PAD,)))


def kernel(A, X, W, b):
    idxp, valp = _score_topk(X, W, b)
    new_A, new_X = _sc_gather(A, X, idxp, valp)
    idx = idxp[:, 0, :N_OUT].astype(jnp.int64)
    return (new_A, new_X, idx)


# speedup vs baseline: 1.0567x; 1.0567x over previous
import functools

import jax
import jax.numpy as jnp
from jax import lax
from jax.experimental import pallas as pl
from jax.experimental.pallas import tpu as pltpu
from jax.experimental.pallas import tpu_sc as plsc

B = 8
N = 2048
D = 128
NUM_QUERIES = 128
NUM_SUPPORTS = N - NUM_QUERIES
K_KEEP = NUM_SUPPORTS // 2
N_OUT = K_KEEP + NUM_QUERIES
N_PAD = 1152

_SUP_BLOCKS = NUM_SUPPORTS // 128

_NW = 32
_WPB = _NW // B
_RPW = N_OUT // _WPB
_CH = 8
_NCHUNK = _RPW // _CH


_JW = 384
_IW = 384


def _score_topk_body(x_ref, w_ref, b_ref, idx_ref, val_ref,
                     sj_ref, si_ref, rank_ref):
    x = x_ref[0]
    w = w_ref[...]
    logits = jnp.dot(x, w, preferred_element_type=jnp.float32)
    scores_col = jax.nn.sigmoid((logits + b_ref[0, 0]) / 100.0)
    scores_row = jnp.reshape(scores_col, (N // 128, 128))

    s_i = scores_col[:NUM_SUPPORTS]
    si_ref[...] = s_i
    sj_ref[...] = jnp.reshape(s_i, (NUM_SUPPORTS // _JW, _JW))
    i_iota = lax.broadcasted_iota(jnp.int32, (NUM_SUPPORTS, 1), 0)

    def cnt_body(t, cnt):
        s_j = sj_ref[pl.ds(t, 1), :]
        j_iota = t * _JW + lax.broadcasted_iota(jnp.int32, (1, _JW), 1)
        gt = s_j > s_i
        eq_lt = (s_j == s_i) & (j_iota < i_iota)
        return cnt + jnp.sum(jnp.where(gt | eq_lt, 1.0, 0.0),
                             axis=1, keepdims=True)

    cnt = lax.fori_loop(0, NUM_SUPPORTS // _JW, cnt_body,
                        jnp.zeros((NUM_SUPPORTS, 1), jnp.float32))
    rank_ref[...] = jnp.where(cnt < K_KEEP, cnt, -1.0)

    r_row = lax.broadcasted_iota(jnp.int32, (1, N_PAD), 1).astype(jnp.float32)

    def scat_body(t, accs):
        idx_acc, val_acc = accs
        rk = rank_ref[pl.ds(t * _IW, _IW), :]
        sv = si_ref[pl.ds(t * _IW, _IW), :]
        iv = (t * _IW
              + lax.broadcasted_iota(jnp.int32, (_IW, 1), 0)).astype(jnp.float32)
        onehot = jnp.where(rk == r_row, 1.0, 0.0)
        idx_acc = idx_acc + jnp.sum(onehot * iv, axis=0, keepdims=True)
        val_acc = val_acc + jnp.sum(onehot * sv, axis=0, keepdims=True)
        return (idx_acc, val_acc)

    idx_acc, val_acc = lax.fori_loop(
        0, NUM_SUPPORTS // _IW, scat_body,
        (jnp.zeros((1, N_PAD), jnp.float32), jnp.zeros((1, N_PAD), jnp.float32)))

    q_scores = scores_row[N // 128 - 1:N // 128, :]
    q_mask = (r_row >= float(K_KEEP)) & (r_row < float(N_OUT))
    q_idx = jnp.where(q_mask, r_row + float(K_KEEP), 0.0)
    q_val_pad = jnp.concatenate(
        [jnp.zeros((1, K_KEEP), jnp.float32), q_scores,
         jnp.zeros((1, N_PAD - N_OUT), jnp.float32)], axis=1)

    idx_ref[...] = jnp.reshape((idx_acc + q_idx).astype(jnp.int32), (1, 1, N_PAD))
    val_ref[...] = jnp.reshape(val_acc + q_val_pad, (1, 1, N_PAD))


def _score_topk(X, W, b):
    w_col = jnp.reshape(W, (D, 1))
    b_mat = jnp.reshape(b, (1, 1))
    idxp, valp = pl.pallas_call(
        _score_topk_body,
        grid=(B,),
        in_specs=[
            pl.BlockSpec((1, N, D), lambda i: (i, 0, 0)),
            pl.BlockSpec((D, 1), lambda i: (0, 0)),
            pl.BlockSpec((1, 1), lambda i: (0, 0)),
        ],
        out_specs=[
            pl.BlockSpec((1, 1, N_PAD), lambda i: (i, 0, 0)),
            pl.BlockSpec((1, 1, N_PAD), lambda i: (i, 0, 0)),
        ],
        out_shape=[
            jax.ShapeDtypeStruct((B, 1, N_PAD), jnp.int32),
            jax.ShapeDtypeStruct((B, 1, N_PAD), jnp.float32),
        ],
        scratch_shapes=[
            pltpu.VMEM((NUM_SUPPORTS // _JW, _JW), jnp.float32),
            pltpu.VMEM((NUM_SUPPORTS, 1), jnp.float32),
            pltpu.VMEM((NUM_SUPPORTS, 1), jnp.float32),
        ],
    )(X, w_col, b_mat)
    return idxp, valp


def _sc_gather_body(a_hbm, x_hbm, idx_hbm, val_hbm, outa_hbm, outx_hbm,
                    colidx_v, rowidx_v, vals_v,
                    abuf0, abuf1, obuf0, obuf1, xbuf0, xbuf1, xobuf0, xobuf1,
                    sem_a0, sem_a1, sem_x0, sem_x1,
                    sem_oa0, sem_oa1, sem_ox0, sem_ox1):
    wid = lax.axis_index("s") * 2 + lax.axis_index("c")
    bi = wid // _WPB
    rb = (wid % _WPB) * _RPW

    base = pl.multiple_of(bi * N_PAD, N_PAD)
    pltpu.sync_copy(idx_hbm.at[pl.ds(base, K_KEEP)], colidx_v)
    pltpu.sync_copy(idx_hbm.at[pl.ds(base + rb, _RPW)], rowidx_v)
    pltpu.sync_copy(val_hbm.at[pl.ds(base + rb, _RPW)], vals_v)

    def rows_of(c):
        return rowidx_v.at[pl.ds(pl.multiple_of(c * _CH, _CH), _CH)]

    def start_in(c, ab, xb, sa, sx):
        pltpu.async_copy(a_hbm.at[bi].at[rows_of(c)], ab, sa)
        pltpu.async_copy(x_hbm.at[bi].at[rows_of(c)], xb, sx)

    def wait_in(c, ab, xb, sa, sx):
        pltpu.make_async_copy(a_hbm.at[bi].at[rows_of(c)], ab, sa).wait()
        pltpu.make_async_copy(x_hbm.at[bi].at[rows_of(c)], xb, sx).wait()

    def out_slices(c):
        c8 = pl.multiple_of(c * _CH, _CH)
        return (outa_hbm.at[bi, pl.ds(rb + c8, _CH), :],
                outx_hbm.at[bi, pl.ds(rb + c8, _CH), :])

    def start_out(c, ob, xob, soa, sox):
        oa, ox = out_slices(c)
        pltpu.async_copy(ob, oa, soa)
        pltpu.async_copy(xob, ox, sox)

    def wait_out(c, ob, xob, soa, sox):
        oa, ox = out_slices(c)
        pltpu.make_async_copy(ob, oa, soa).wait()
        pltpu.make_async_copy(xob, ox, sox).wait()

    def compute(c, ab, xb, ob, xob):
        c8 = pl.multiple_of(c * _CH, _CH)
        for g in range(K_KEEP // 16):
            ci = colidx_v[pl.ds(g * 16, 16)]
            for j in range(_CH):
                ob[j, pl.ds(g * 16, 16)] = plsc.load_gather(ab.at[j], [ci])
        for j in range(_CH):
            for g in range(NUM_QUERIES // 16):
                ob[j, pl.ds(K_KEEP + g * 16, 16)] = \
                    ab[j, pl.ds(NUM_SUPPORTS + g * 16, 16)]
            vsp = plsc.load_gather(vals_v, [jnp.full((16,), 1, jnp.int32)
                                            * (c8 + j)])
            for g in range(D // 16):
                xob[j, pl.ds(g * 16, 16)] = xb[j, pl.ds(g * 16, 16)] * vsp

    bufs = ((abuf0, xbuf0, obuf0, xobuf0, sem_a0, sem_x0, sem_oa0, sem_ox0),
            (abuf1, xbuf1, obuf1, xobuf1, sem_a1, sem_x1, sem_oa1, sem_ox1))

    start_in(0, abuf0, xbuf0, sem_a0, sem_x0)

    def pair(i, carry):
        c0 = i * 2
        for par in range(2):
            ab, xb, ob, xob, sa, sx, soa, sox = bufs[par]
            nab, nxb, _, _, nsa, nsx, _, _ = bufs[1 - par]
            c = c0 + par

            @pl.when(c + 1 < _NCHUNK)
            def _():
                start_in(c + 1, nab, nxb, nsa, nsx)

            wait_in(c, ab, xb, sa, sx)

            @pl.when(c >= 2)
            def _():
                wait_out(c - 2, ob, xob, soa, sox)

            compute(c, ab, xb, ob, xob)
            start_out(c, ob, xob, soa, sox)
        return carry

    lax.fori_loop(0, _NCHUNK // 2, pair, 0)
    if _NCHUNK % 2:
        c = _NCHUNK - 1
        wait_in(c, abuf0, xbuf0, sem_a0, sem_x0)
        wait_out(c - 2, obuf0, xobuf0, sem_oa0, sem_ox0)
        compute(c, abuf0, xbuf0, obuf0, xobuf0)
        start_out(c, obuf0, xobuf0, sem_oa0, sem_ox0)
    wait_out(_NCHUNK - 2, obuf1 if _NCHUNK % 2 else obuf0,
             xobuf1 if _NCHUNK % 2 else xobuf0,
             sem_oa1 if _NCHUNK % 2 else sem_oa0,
             sem_ox1 if _NCHUNK % 2 else sem_ox0)
    wait_out(_NCHUNK - 1, obuf0 if _NCHUNK % 2 else obuf1,
             xobuf0 if _NCHUNK % 2 else xobuf1,
             sem_oa0 if _NCHUNK % 2 else sem_oa1,
             sem_ox0 if _NCHUNK % 2 else sem_ox1)


def _sc_gather(A, X, idxp, valp):
    mesh = plsc.VectorSubcoreMesh(core_axis_name="c", subcore_axis_name="s")
    kern = pl.kernel(
        _sc_gather_body,
        out_type=[
            jax.ShapeDtypeStruct((B, N_OUT, N_OUT), jnp.float32),
            jax.ShapeDtypeStruct((B, N_OUT, D), jnp.float32),
        ],
        mesh=mesh,
        scratch_types=(
            [pltpu.VMEM((K_KEEP,), jnp.int32),
             pltpu.VMEM((_RPW,), jnp.int32),
             pltpu.VMEM((_RPW,), jnp.float32)]
            + [pltpu.VMEM((_CH, N), jnp.float32)] * 2
            + [pltpu.VMEM((_CH, N_OUT), jnp.float32)] * 2
            + [pltpu.VMEM((_CH, D), jnp.float32)] * 4
            + [pltpu.SemaphoreType.DMA] * 8
        ),
        compiler_params=pltpu.CompilerParams(use_tc_tiling_on_sc=False,
                                             needs_layout_passes=False),
    )
    return kern(A, X, jnp.reshape(idxp, (B * N_PAD,)),
                jnp.reshape(valp, (B * N_PAD,)))


def kernel(A, X, W, b):
    idxp, valp = _score_topk(X, W, b)
    new_A, new_X = _sc_gather(A, X, idxp, valp)
    idx = idxp[:, 0, :N_OUT].astype(jnp.int64)
    return (new_A, new_X, idx)


# speedup vs baseline: 1.0595x; 1.0026x over previous
import functools

import jax
import jax.numpy as jnp
from jax import lax
from jax.experimental import pallas as pl
from jax.experimental.pallas import tpu as pltpu
from jax.experimental.pallas import tpu_sc as plsc

B = 8
N = 2048
D = 128
NUM_QUERIES = 128
NUM_SUPPORTS = N - NUM_QUERIES
K_KEEP = NUM_SUPPORTS // 2
N_OUT = K_KEEP + NUM_QUERIES
N_PAD = 1152

_SUP_BLOCKS = NUM_SUPPORTS // 128

_NW = 32
_WPB = _NW // B
_RPW = N_OUT // _WPB
_CH = 8
_NCHUNK = _RPW // _CH


_JW = 384
_IW = 384


def _score_topk_body(x_ref, w_ref, b_ref, idx_ref, val_ref,
                     sj_ref, si_ref, rank_ref):
    x = x_ref[0]
    w = w_ref[...]
    logits = jnp.dot(x, w, preferred_element_type=jnp.float32)
    scores_col = jax.nn.sigmoid((logits + b_ref[0, 0]) / 100.0)
    scores_row = jnp.reshape(scores_col, (N // 128, 128))

    s_i = scores_col[:NUM_SUPPORTS]
    si_ref[...] = s_i
    sj_ref[...] = jnp.reshape(s_i, (NUM_SUPPORTS // _JW, _JW))
    i_iota = lax.broadcasted_iota(jnp.int32, (NUM_SUPPORTS, 1), 0)

    def cnt_body(t, cnt):
        s_j = sj_ref[pl.ds(t, 1), :]
        j_iota = t * _JW + lax.broadcasted_iota(jnp.int32, (1, _JW), 1)
        gt = s_j > s_i
        eq_lt = (s_j == s_i) & (j_iota < i_iota)
        return cnt + jnp.sum(jnp.where(gt | eq_lt, 1.0, 0.0),
                             axis=1, keepdims=True)

    cnt = lax.fori_loop(0, NUM_SUPPORTS // _JW, cnt_body,
                        jnp.zeros((NUM_SUPPORTS, 1), jnp.float32))
    rank_ref[...] = jnp.where(cnt < K_KEEP, cnt, -1.0)

    r_row = lax.broadcasted_iota(jnp.int32, (1, N_PAD), 1).astype(jnp.float32)

    def scat_body(t, accs):
        idx_acc, val_acc = accs
        rk = rank_ref[pl.ds(t * _IW, _IW), :]
        sv = si_ref[pl.ds(t * _IW, _IW), :]
        iv = (t * _IW
              + lax.broadcasted_iota(jnp.int32, (_IW, 1), 0)).astype(jnp.float32)
        onehot = jnp.where(rk == r_row, 1.0, 0.0)
        idx_acc = idx_acc + jnp.sum(onehot * iv, axis=0, keepdims=True)
        val_acc = val_acc + jnp.sum(onehot * sv, axis=0, keepdims=True)
        return (idx_acc, val_acc)

    idx_acc, val_acc = lax.fori_loop(
        0, NUM_SUPPORTS // _IW, scat_body,
        (jnp.zeros((1, N_PAD), jnp.float32), jnp.zeros((1, N_PAD), jnp.float32)))

    q_scores = scores_row[N // 128 - 1:N // 128, :]
    q_mask = (r_row >= float(K_KEEP)) & (r_row < float(N_OUT))
    q_idx = jnp.where(q_mask, r_row + float(K_KEEP), 0.0)
    q_val_pad = jnp.concatenate(
        [jnp.zeros((1, K_KEEP), jnp.float32), q_scores,
         jnp.zeros((1, N_PAD - N_OUT), jnp.float32)], axis=1)

    idx_ref[...] = jnp.reshape((idx_acc + q_idx).astype(jnp.int32), (1, 1, N_PAD))
    val_ref[...] = jnp.reshape(val_acc + q_val_pad, (1, 1, N_PAD))


def _score_topk(X, W, b):
    w_col = jnp.reshape(W, (D, 1))
    b_mat = jnp.reshape(b, (1, 1))
    idxp, valp = pl.pallas_call(
        _score_topk_body,
        grid=(B,),
        in_specs=[
            pl.BlockSpec((1, N, D), lambda i: (i, 0, 0)),
            pl.BlockSpec((D, 1), lambda i: (0, 0)),
            pl.BlockSpec((1, 1), lambda i: (0, 0)),
        ],
        out_specs=[
            pl.BlockSpec((1, 1, N_PAD), lambda i: (i, 0, 0)),
            pl.BlockSpec((1, 1, N_PAD), lambda i: (i, 0, 0)),
        ],
        out_shape=[
            jax.ShapeDtypeStruct((B, 1, N_PAD), jnp.int32),
            jax.ShapeDtypeStruct((B, 1, N_PAD), jnp.float32),
        ],
        scratch_shapes=[
            pltpu.VMEM((NUM_SUPPORTS // _JW, _JW), jnp.float32),
            pltpu.VMEM((NUM_SUPPORTS, 1), jnp.float32),
            pltpu.VMEM((NUM_SUPPORTS, 1), jnp.float32),
        ],
    )(X, w_col, b_mat)
    return idxp, valp


def _sc_gather_body(a_hbm, x_hbm, idx_hbm, val_hbm, outa_hbm, outx_hbm,
                    colidx_v, rowidx_v, vals_v,
                    abuf0, abuf1, obuf0, obuf1, xbuf0, xbuf1, xobuf0, xobuf1,
                    sem_a0, sem_a1, sem_x0, sem_x1,
                    sem_oa0, sem_oa1, sem_ox0, sem_ox1, sem_q0, sem_q1):
    wid = lax.axis_index("s") * 2 + lax.axis_index("c")
    bi = wid // _WPB
    rb = (wid % _WPB) * _RPW

    base = pl.multiple_of(bi * N_PAD, N_PAD)
    pltpu.sync_copy(idx_hbm.at[pl.ds(base, K_KEEP)], colidx_v)
    pltpu.sync_copy(idx_hbm.at[pl.ds(base + rb, _RPW)], rowidx_v)
    pltpu.sync_copy(val_hbm.at[pl.ds(base + rb, _RPW)], vals_v)

    def rows_of(c):
        return rowidx_v.at[pl.ds(pl.multiple_of(c * _CH, _CH), _CH)]

    def start_in(c, ab, xb, sa, sx):
        pltpu.async_copy(a_hbm.at[bi].at[rows_of(c)], ab, sa)
        pltpu.async_copy(x_hbm.at[bi].at[rows_of(c)], xb, sx)

    def wait_in(c, ab, xb, sa, sx):
        pltpu.make_async_copy(a_hbm.at[bi].at[rows_of(c)], ab, sa).wait()
        pltpu.make_async_copy(x_hbm.at[bi].at[rows_of(c)], xb, sx).wait()

    def out_slices(c):
        c8 = pl.multiple_of(c * _CH, _CH)
        return (outa_hbm.at[bi, pl.ds(rb + c8, _CH), pl.ds(0, K_KEEP)],
                outx_hbm.at[bi, pl.ds(rb + c8, _CH), :])

    def q_slices(c, ab):
        c8 = pl.multiple_of(c * _CH, _CH)
        return (ab.at[:, pl.ds(NUM_SUPPORTS, NUM_QUERIES)],
                outa_hbm.at[bi, pl.ds(rb + c8, _CH),
                            pl.ds(K_KEEP, NUM_QUERIES)])

    def start_q(c, ab, sq):
        src, dst = q_slices(c, ab)
        pltpu.async_copy(src, dst, sq)

    def wait_q(c, ab, sq):
        src, dst = q_slices(c, ab)
        pltpu.make_async_copy(src, dst, sq).wait()

    def start_out(c, ob, xob, soa, sox):
        oa, ox = out_slices(c)
        pltpu.async_copy(ob, oa, soa)
        pltpu.async_copy(xob, ox, sox)

    def wait_out(c, ob, xob, soa, sox):
        oa, ox = out_slices(c)
        pltpu.make_async_copy(ob, oa, soa).wait()
        pltpu.make_async_copy(xob, ox, sox).wait()

    def compute(c, ab, xb, ob, xob):
        c8 = pl.multiple_of(c * _CH, _CH)
        for g in range(K_KEEP // 16):
            ci = colidx_v[pl.ds(g * 16, 16)]
            for j in range(_CH):
                ob[j, pl.ds(g * 16, 16)] = plsc.load_gather(ab.at[j], [ci])
        for j in range(_CH):
            vsp = plsc.load_gather(vals_v, [jnp.full((16,), 1, jnp.int32)
                                            * (c8 + j)])
            for g in range(D // 16):
                xob[j, pl.ds(g * 16, 16)] = xb[j, pl.ds(g * 16, 16)] * vsp

    bufs = ((abuf0, xbuf0, obuf0, xobuf0,
             sem_a0, sem_x0, sem_oa0, sem_ox0, sem_q0),
            (abuf1, xbuf1, obuf1, xobuf1,
             sem_a1, sem_x1, sem_oa1, sem_ox1, sem_q1))

    start_in(0, abuf0, xbuf0, sem_a0, sem_x0)

    def pair(i, carry):
        c0 = i * 2
        for par in range(2):
            ab, xb, ob, xob, sa, sx, soa, sox, sq = bufs[par]
            nab, nxb, _, _, nsa, nsx, _, _, nsq = bufs[1 - par]
            c = c0 + par

            @pl.when(c >= 1)
            def _():
                wait_q(c - 1, nab, nsq)

            @pl.when(c + 1 < _NCHUNK)
            def _():
                start_in(c + 1, nab, nxb, nsa, nsx)

            wait_in(c, ab, xb, sa, sx)

            @pl.when(c >= 2)
            def _():
                wait_out(c - 2, ob, xob, soa, sox)

            compute(c, ab, xb, ob, xob)
            start_out(c, ob, xob, soa, sox)
            start_q(c, ab, sq)
        return carry

    lax.fori_loop(0, _NCHUNK // 2, pair, 0)
    wait_q(_NCHUNK - 1, abuf1 if (_NCHUNK - 1) % 2 else abuf0,
           sem_q1 if (_NCHUNK - 1) % 2 else sem_q0)
    if _NCHUNK % 2:
        c = _NCHUNK - 1
        wait_in(c, abuf0, xbuf0, sem_a0, sem_x0)
        wait_out(c - 2, obuf0, xobuf0, sem_oa0, sem_ox0)
        compute(c, abuf0, xbuf0, obuf0, xobuf0)
        start_out(c, obuf0, xobuf0, sem_oa0, sem_ox0)
    wait_out(_NCHUNK - 2, obuf1 if _NCHUNK % 2 else obuf0,
             xobuf1 if _NCHUNK % 2 else xobuf0,
             sem_oa1 if _NCHUNK % 2 else sem_oa0,
             sem_ox1 if _NCHUNK % 2 else sem_ox0)
    wait_out(_NCHUNK - 1, obuf0 if _NCHUNK % 2 else obuf1,
             xobuf0 if _NCHUNK % 2 else xobuf1,
             sem_oa0 if _NCHUNK % 2 else sem_oa1,
             sem_ox0 if _NCHUNK % 2 else sem_ox1)


def _sc_gather(A, X, idxp, valp):
    mesh = plsc.VectorSubcoreMesh(core_axis_name="c", subcore_axis_name="s")
    kern = pl.kernel(
        _sc_gather_body,
        out_type=[
            jax.ShapeDtypeStruct((B, N_OUT, N_OUT), jnp.float32),
            jax.ShapeDtypeStruct((B, N_OUT, D), jnp.float32),
        ],
        mesh=mesh,
        scratch_types=(
            [pltpu.VMEM((K_KEEP,), jnp.int32),
             pltpu.VMEM((_RPW,), jnp.int32),
             pltpu.VMEM((_RPW,), jnp.float32)]
            + [pltpu.VMEM((_CH, N), jnp.float32)] * 2
            + [pltpu.VMEM((_CH, K_KEEP), jnp.float32)] * 2
            + [pltpu.VMEM((_CH, D), jnp.float32)] * 4
            + [pltpu.SemaphoreType.DMA] * 10
        ),
        compiler_params=pltpu.CompilerParams(use_tc_tiling_on_sc=False,
                                             needs_layout_passes=False),
    )
    return kern(A, X, jnp.reshape(idxp, (B * N_PAD,)),
                jnp.reshape(valp, (B * N_PAD,)))


def kernel(A, X, W, b):
    idxp, valp = _score_topk(X, W, b)
    new_A, new_X = _sc_gather(A, X, idxp, valp)
    idx = idxp[:, 0, :N_OUT].astype(jnp.int64)
    return (new_A, new_X, idx)
